# baseline (speedup 1.0000x reference)
import functools

import jax
import jax.numpy as jnp
from jax import lax
from jax.experimental import pallas as pl
from jax.experimental.pallas import tpu as pltpu
from jax.experimental.pallas import tpu_sc as plsc

N_NODES = 100000
NUM_EMB = 21
HIDDEN = 128

NC = 2
NS = 16
NW = NC * NS

PER_W = 3128
LAST_BASE = N_NODES - PER_W
CHUNK = 128
FULL = PER_W // CHUNK
TAIL = PER_W - FULL * CHUNK
CHUNKS = FULL + 1

NBUF = 4

_mesh = plsc.VectorSubcoreMesh(core_axis_name="c", subcore_axis_name="s")


@functools.partial(
    pl.kernel,
    mesh=_mesh,
    out_type=jax.ShapeDtypeStruct((N_NODES, HIDDEN), jnp.float32),
    scratch_types=[
        pltpu.VMEM_SHARED((NUM_EMB, HIDDEN), jnp.float32),
        pltpu.VMEM((PER_W,), jnp.int32),
        pltpu.VMEM((NBUF, CHUNK, HIDDEN), jnp.float32),
        pltpu.SemaphoreType.DMA((NBUF,)),
        pltpu.SemaphoreType.DMA((NBUF,)),
    ],
)
def _emb_lookup(idx_hbm, table_hbm, out_hbm, table_sh, idx_v, rows_v, gsem, wsem):
    sid = lax.axis_index("s")
    wid = sid * NC + lax.axis_index("c")
    base = lax.min(wid * PER_W, LAST_BASE)

    @pl.when(sid == 0)
    def _stage_table():
        pltpu.sync_copy(table_hbm, table_sh)

    pltpu.sync_copy(idx_hbm.at[pl.ds(base, PER_W)], idx_v)
    plsc.subcore_barrier()

    def gather(c, b):
        off = pl.multiple_of(c * CHUNK, CHUNK)
        return pltpu.make_async_copy(
            table_sh.at[idx_v.at[pl.ds(off, CHUNK)]], rows_v.at[b], gsem.at[b])

    def write(c, b):
        off = pl.multiple_of(c * CHUNK, CHUNK)
        return pltpu.make_async_copy(
            rows_v.at[b], out_hbm.at[pl.ds(base + off, CHUNK)], wsem.at[b])

    for c in range(NBUF):
        gather(c, c).start()
        if c >= 3:
            gather(c - 3, c - 3).wait()
            write(c - 3, c - 3).start()

    def slot(c, carry):
        b = lax.rem(c, NBUF)
        pb = lax.rem(c + (NBUF - 3), NBUF)
        write(c - NBUF, b).wait()
        gather(c, b).start()
        gather(c - 3, pb).wait()
        write(c - 3, pb).start()
        return carry

    lax.fori_loop(NBUF, FULL, slot, 0)

    for c in range(FULL - 3, FULL):
        gather(c, c % NBUF).wait()
        write(c, c % NBUF).start()
    write(FULL - NBUF, 0).wait()
    t_off = FULL * CHUNK
    tg = pltpu.make_async_copy(
        table_sh.at[idx_v.at[pl.ds(t_off, TAIL)]],
        rows_v.at[0, pl.ds(0, TAIL)], gsem.at[0])
    tg.start()
    tg.wait()
    tw = pltpu.make_async_copy(
        rows_v.at[0, pl.ds(0, TAIL)],
        out_hbm.at[pl.ds(base + t_off, TAIL)], wsem.at[0])
    tw.start()
    for c in range(FULL - NBUF + 1, FULL):
        write(c, c % NBUF).wait()
    tw.wait()


def kernel(x, emb_weight):
    return _emb_lookup(x.astype(jnp.int32), emb_weight)


# speedup vs baseline: 1.0034x; 1.0034x over previous
import functools

import jax
import jax.numpy as jnp
from jax import lax
from jax.experimental import pallas as pl
from jax.experimental.pallas import tpu as pltpu
from jax.experimental.pallas import tpu_sc as plsc

N_NODES = 100000
NUM_EMB = 21
HIDDEN = 128

NC = 2
NS = 16
NW = NC * NS

PER_W = 3128
LAST_BASE = N_NODES - PER_W
CHUNK = 128
FULL = PER_W // CHUNK
TAIL = PER_W - FULL * CHUNK
CHUNKS = FULL + 1

NBUF = 4

_mesh = plsc.VectorSubcoreMesh(core_axis_name="c", subcore_axis_name="s")


@functools.partial(
    pl.kernel,
    mesh=_mesh,
    out_type=jax.ShapeDtypeStruct((N_NODES, HIDDEN), jnp.float32),
    scratch_types=[
        pltpu.VMEM_SHARED((NUM_EMB, HIDDEN), jnp.float32),
        pltpu.VMEM((PER_W,), jnp.int32),
        pltpu.VMEM((NBUF, CHUNK, HIDDEN), jnp.float32),
        pltpu.SemaphoreType.DMA((NBUF,)),
        pltpu.SemaphoreType.DMA((NBUF,)),
    ],
)
def _emb_lookup(idx_hbm, table_hbm, out_hbm, table_sh, idx_v, rows_v, gsem, wsem):
    sid = lax.axis_index("s")
    wid = sid * NC + lax.axis_index("c")
    base = lax.min(wid * PER_W, LAST_BASE)

    @pl.when(sid == 0)
    def _stage_table():
        pltpu.sync_copy(table_hbm, table_sh)

    pltpu.sync_copy(idx_hbm.at[pl.ds(base, PER_W)], idx_v)
    plsc.subcore_barrier()

    def gather(c, b):
        off = pl.multiple_of(c * CHUNK, CHUNK)
        return pltpu.make_async_copy(
            table_sh.at[idx_v.at[pl.ds(off, CHUNK)]], rows_v.at[b], gsem.at[b])

    def write(c, b):
        off = pl.multiple_of(c * CHUNK, CHUNK)
        return pltpu.make_async_copy(
            rows_v.at[b], out_hbm.at[pl.ds(base + off, CHUNK)], wsem.at[b])

    for c in range(NBUF):
        gather(c, c).start()
        if c >= 2:
            gather(c - 2, c - 2).wait()
            write(c - 2, c - 2).start()

    def slot(c, carry):
        b = lax.rem(c, NBUF)
        pb = lax.rem(c + (NBUF - 2), NBUF)
        write(c - NBUF, b).wait()
        gather(c, b).start()
        gather(c - 2, pb).wait()
        write(c - 2, pb).start()
        return carry

    lax.fori_loop(NBUF, FULL, slot, 0)

    for c in range(FULL - 2, FULL):
        gather(c, c % NBUF).wait()
        write(c, c % NBUF).start()
    write(FULL - NBUF, 0).wait()
    t_off = FULL * CHUNK
    tg = pltpu.make_async_copy(
        table_sh.at[idx_v.at[pl.ds(t_off, TAIL)]],
        rows_v.at[0, pl.ds(0, TAIL)], gsem.at[0])
    tg.start()
    tg.wait()
    tw = pltpu.make_async_copy(
        rows_v.at[0, pl.ds(0, TAIL)],
        out_hbm.at[pl.ds(base + t_off, TAIL)], wsem.at[0])
    tw.start()
    for c in range(FULL - NBUF + 1, FULL):
        write(c, c % NBUF).wait()
    tw.wait()


def kernel(x, emb_weight):
    return _emb_lookup(x.astype(jnp.int32), emb_weight)


# speedup vs baseline: 1.0244x; 1.0210x over previous
import functools

import jax
import jax.numpy as jnp
from jax import lax
from jax.experimental import pallas as pl
from jax.experimental.pallas import tpu as pltpu
from jax.experimental.pallas import tpu_sc as plsc

N_NODES = 100000
NUM_EMB = 21
HIDDEN = 128

NC = 2
NS = 16
NW = NC * NS

PER_W = 3128
LAST_BASE = N_NODES - PER_W
CHUNK = 128
FULL = PER_W // CHUNK
TAIL = PER_W - FULL * CHUNK
CHUNKS = FULL + 1

NBUF = 4

_mesh = plsc.VectorSubcoreMesh(core_axis_name="c", subcore_axis_name="s")


@functools.partial(
    pl.kernel,
    mesh=_mesh,
    out_type=jax.ShapeDtypeStruct((N_NODES, HIDDEN), jnp.float32),
    scratch_types=[
        pltpu.VMEM_SHARED((NUM_EMB, HIDDEN), jnp.float32),
        pltpu.VMEM((PER_W,), jnp.int32),
        pltpu.VMEM((NBUF, CHUNK, HIDDEN), jnp.float32),
        pltpu.VMEM((TAIL, HIDDEN), jnp.float32),
        pltpu.SemaphoreType.DMA((NBUF,)),
        pltpu.SemaphoreType.DMA((NBUF,)),
        pltpu.SemaphoreType.DMA,
    ],
)
def _emb_lookup(idx_hbm, table_hbm, out_hbm, table_sh, idx_v, rows_v, tail_v,
                gsem, wsem, tsem):
    sid = lax.axis_index("s")
    wid = sid * NC + lax.axis_index("c")
    base = lax.min(wid * PER_W, LAST_BASE)

    icopy = pltpu.make_async_copy(idx_hbm.at[pl.ds(base, PER_W)], idx_v, gsem.at[0])
    icopy.start()

    @pl.when(sid == 0)
    def _stage_table():
        tc = pltpu.make_async_copy(table_hbm, table_sh, tsem)
        tc.start()
        tc.wait()

    icopy.wait()
    plsc.subcore_barrier()

    def gather(c, b):
        off = pl.multiple_of(c * CHUNK, CHUNK)
        return pltpu.make_async_copy(
            table_sh.at[idx_v.at[pl.ds(off, CHUNK)]], rows_v.at[b], gsem.at[b])

    def write(c, b):
        off = pl.multiple_of(c * CHUNK, CHUNK)
        return pltpu.make_async_copy(
            rows_v.at[b], out_hbm.at[pl.ds(base + off, CHUNK)], wsem.at[b])

    t_off = FULL * CHUNK
    tg = pltpu.make_async_copy(
        table_sh.at[idx_v.at[pl.ds(t_off, TAIL)]], tail_v, tsem)
    tg.start()
    for c in range(NBUF):
        gather(c, c).start()
        if c >= 2:
            gather(c - 2, c - 2).wait()
            write(c - 2, c - 2).start()

    def slot(c, carry):
        b = lax.rem(c, NBUF)
        pb = lax.rem(c + (NBUF - 2), NBUF)
        write(c - NBUF, b).wait()
        gather(c, b).start()
        gather(c - 2, pb).wait()
        write(c - 2, pb).start()
        return carry

    lax.fori_loop(NBUF, FULL, slot, 0)

    tg.wait()
    tw = pltpu.make_async_copy(
        tail_v, out_hbm.at[pl.ds(base + t_off, TAIL)], tsem)
    tw.start()
    for c in range(FULL - 2, FULL):
        gather(c, c % NBUF).wait()
        write(c, c % NBUF).start()
    for c in range(FULL - NBUF, FULL):
        write(c, c % NBUF).wait()
    tw.wait()


def kernel(x, emb_weight):
    return _emb_lookup(x.astype(jnp.int32), emb_weight)
